# baseline (speedup 1.0000x reference)
import functools

import jax
import jax.numpy as jnp
from jax import lax
from jax.experimental import pallas as pl
from jax.experimental.pallas import tpu as pltpu
from jax.experimental.pallas import tpu_sc as plsc

R = 64
N = 32768
K = 16
L = 16
GV = 8
GROUPS = N // (GV * L)
SB = GROUPS // L
ROWS_PER_W = 2

_NEG_INF = float("-inf")


def _dyn_gather(src, idx):
    dn = lax.GatherDimensionNumbers(
        offset_dims=(), collapsed_slice_dims=(0,), start_index_map=(0,))
    return lax.gather(src, idx[:, None], dn, (1,),
                      mode=lax.GatherScatterMode.PROMISE_IN_BOUNDS)


def _splat(x, dtype=jnp.int32):
    return jnp.broadcast_to(jnp.asarray(x, dtype), (L,))


def _rotate_reduce(x, lane, op):
    for k in (1, 2, 4, 8):
        x = op(x, _dyn_gather(x, jnp.bitwise_and(lane + k, L - 1)))
    return x


def _process_row(row_ref, slot, cand_ref, summ_ref, lane, mid_wait=None):
    islot = _splat(slot)
    last = lane == (L - 1)

    def pass_a(sb, _):
        for g in range(L):
            base = sb * (L * GV * L) + g * (GV * L)
            x = row_ref[slot, pl.ds(base, L)]
            for j in range(1, GV):
                x = jnp.maximum(x, row_ref[slot, pl.ds(base + j * L, L)])
            plsc.store_scatter(summ_ref, [_splat(sb * L + g)],
                               plsc.cummax(x), mask=last)
        return 0

    lax.fori_loop(0, SB // 2, pass_a, 0)
    if mid_wait is not None:
        mid_wait()
    lax.fori_loop(SB // 2, SB, pass_a, 0)

    col = summ_ref[pl.ds(0, L)]
    for sb in range(1, SB):
        col = jnp.maximum(col, summ_ref[pl.ds(sb * L, L)])
    t0 = _rotate_reduce(col, lane, jnp.minimum)

    def pass_c(sb, off):
        s = summ_ref[pl.ds(sb * L, L)]
        nh = lax.reduce_max(
            plsc.all_reduce_population_count(s >= t0), axes=(0,))

        def hit_body(_, c):
            s, off = c
            ffs = plsc.all_reduce_ffs(s >= t0)
            g = lax.reduce_max(ffs, axes=(0,))
            base = sb * (L * GV * L) + g * (GV * L)
            gb = _splat(base)
            mis, ivs, cnts = [], [], []
            for j in range(GV):
                v = row_ref[slot, pl.ds(base + j * L, L)]
                ivec = gb + _splat(j * L) + lane
                m = v >= t0
                mis.append(jnp.where(m, _splat(1), _splat(0)))
                ivs.append(ivec)
                cnts.append(plsc.all_reduce_population_count(m))
            base2 = off
            for j in range(GV):
                pos = base2 + plsc.cumsum(mis[j]) - mis[j]
                plsc.store_scatter(cand_ref, [pos], ivs[j],
                                   mask=mis[j] > 0)
                base2 = base2 + cnts[j]
            s2 = jnp.where(lane == ffs,
                           jnp.full((L,), _NEG_INF, jnp.float32), s)
            return (s2, base2)

        return lax.fori_loop(0, nh, hit_body, (s, off))[1]

    off = lax.fori_loop(0, SB, pass_c, _splat(0))

    cnt = lax.reduce_max(off, axes=(0,))
    T = jnp.full((L,), _NEG_INF, jnp.float32)
    TI = jnp.zeros((L,), jnp.int32)

    def pass_d(k, c):
        T, TI = c
        iv = cand_ref[pl.ds(k * L, L)]
        valid = (_splat(k * L) + lane) < off
        ivc = jnp.minimum(jnp.maximum(iv, 0), N - 1)
        gv = plsc.load_gather(row_ref, [islot, ivc])
        gv = jnp.where(valid, gv, jnp.full((L,), _NEG_INF, jnp.float32))
        ivc = jnp.where(valid, ivc, _splat(0))
        nc = jnp.minimum(cnt - k * L, L)

        def ins_body(_, c2):
            gv, T, TI = c2
            ffs = plsc.all_reduce_ffs(gv >= t0)
            cv = _dyn_gather(gv, ffs)
            ci = _dyn_gather(ivc, ffs)
            icnt = plsc.all_reduce_population_count(T >= cv)
            shT = _dyn_gather(T, jnp.maximum(lane - 1, 0))
            shTI = _dyn_gather(TI, jnp.maximum(lane - 1, 0))
            T2 = jnp.where(lane < icnt, T, jnp.where(lane == icnt, cv, shT))
            TI2 = jnp.where(lane < icnt, TI,
                            jnp.where(lane == icnt, ci, shTI))
            gv2 = jnp.where(lane == ffs,
                            jnp.full((L,), _NEG_INF, jnp.float32), gv)
            return (gv2, T2, TI2)

        _, T, TI = lax.fori_loop(0, nc, ins_body, (gv, T, TI))
        return (T, TI)

    T, TI = lax.fori_loop(0, (cnt + L - 1) // L, pass_d, (T, TI))
    return T, TI


@functools.partial(
    pl.kernel,
    mesh=plsc.VectorSubcoreMesh(core_axis_name="c", subcore_axis_name="s"),
    compiler_params=pltpu.CompilerParams(needs_layout_passes=False,
                                         use_tc_tiling_on_sc=True),
    out_type=[
        jax.ShapeDtypeStruct((R, K), jnp.float32),
        jax.ShapeDtypeStruct((R, K), jnp.int32),
    ],
    scratch_types=[
        pltpu.VMEM((ROWS_PER_W, N), jnp.float32),
        pltpu.VMEM((N,), jnp.int32),
        pltpu.VMEM((GROUPS,), jnp.float32),
        pltpu.VMEM((K,), jnp.float32),
        pltpu.VMEM((K,), jnp.int32),
        pltpu.SemaphoreType.DMA,
        pltpu.SemaphoreType.DMA,
        pltpu.SemaphoreType.DMA,
    ],
)
def _topk_kernel(scores_hbm, vals_hbm, idx_hbm,
                 rows_v, cand_v, summ_v, vals_v, idx_v, sem0, sem1, sem2):
    info = plsc.get_sparse_core_info()
    nc = info.num_cores
    wid = lax.axis_index("s") * nc + lax.axis_index("c")
    r0 = wid * ROWS_PER_W
    lane = lax.iota(jnp.int32, L)
    H = N // 2

    cp0a = pltpu.async_copy(scores_hbm.at[r0, pl.ds(0, H)],
                            rows_v.at[0, pl.ds(0, H)], sem0)
    cp0b = pltpu.async_copy(scores_hbm.at[r0, pl.ds(H, H)],
                            rows_v.at[0, pl.ds(H, H)], sem2)
    cp1 = pltpu.async_copy(scores_hbm.at[r0 + 1], rows_v.at[1], sem1)

    cp0a.wait()
    T, TI = _process_row(rows_v, 0, cand_v, summ_v, lane,
                         mid_wait=cp0b.wait)
    vals_v[...] = T
    idx_v[...] = TI
    pltpu.sync_copy(vals_v, vals_hbm.at[r0])
    pltpu.sync_copy(idx_v, idx_hbm.at[r0])

    cp1.wait()
    T, TI = _process_row(rows_v, 1, cand_v, summ_v, lane)
    vals_v[...] = T
    idx_v[...] = TI
    pltpu.sync_copy(vals_v, vals_hbm.at[r0 + 1])
    pltpu.sync_copy(idx_v, idx_hbm.at[r0 + 1])


def kernel(scores):
    vals, idx = _topk_kernel(scores)
    return vals, idx


# speedup vs baseline: 1.1431x; 1.1431x over previous
import functools

import jax
import jax.numpy as jnp
from jax import lax
from jax.experimental import pallas as pl
from jax.experimental.pallas import tpu as pltpu
from jax.experimental.pallas import tpu_sc as plsc

R = 64
N = 32768
K = 16
L = 16
GV = 8
GROUPS = N // (GV * L)
SB = GROUPS // L
ROWS_PER_W = 2

_NEG_INF = float("-inf")


def _dyn_gather(src, idx):
    dn = lax.GatherDimensionNumbers(
        offset_dims=(), collapsed_slice_dims=(0,), start_index_map=(0,))
    return lax.gather(src, idx[:, None], dn, (1,),
                      mode=lax.GatherScatterMode.PROMISE_IN_BOUNDS)


def _splat(x, dtype=jnp.int32):
    return jnp.broadcast_to(jnp.asarray(x, dtype), (L,))


def _rotate_reduce(x, lane, op):
    for k in (1, 2, 4, 8):
        x = op(x, _dyn_gather(x, jnp.bitwise_and(lane + k, L - 1)))
    return x


def _process_row(row_ref, slot, cand_ref, summ_ref, lane):
    islot = _splat(slot)

    def pass_a(sb, _):
        acc = jnp.full((L,), _NEG_INF, jnp.float32)
        for g in range(L):
            base = sb * (L * GV * L) + g * (GV * L)
            x = row_ref[slot, pl.ds(base, L)]
            for j in range(1, GV):
                x = jnp.maximum(x, row_ref[slot, pl.ds(base + j * L, L)])
            gm = _dyn_gather(plsc.cummax(x), _splat(K - 1))
            acc = jnp.where(lane == g, gm, acc)
        summ_ref[pl.ds(sb * L, L)] = acc
        return 0

    lax.fori_loop(0, SB, pass_a, 0)

    col = summ_ref[pl.ds(0, L)]
    for sb in range(1, SB):
        col = jnp.maximum(col, summ_ref[pl.ds(sb * L, L)])
    t0 = _rotate_reduce(col, lane, jnp.minimum)

    def pass_c(sb, off):
        s = summ_ref[pl.ds(sb * L, L)]
        nh = lax.reduce_max(
            plsc.all_reduce_population_count(s >= t0), axes=(0,))

        def hit_body(_, c):
            s, off = c
            ffs = plsc.all_reduce_ffs(s >= t0)
            gb = _splat(sb * (L * GV * L)) + (ffs << 7)
            mis, ivs, cnts = [], [], []
            for j in range(GV):
                ivec = gb + _splat(j * L) + lane
                v = plsc.load_gather(row_ref, [islot, ivec])
                m = v >= t0
                mis.append(jnp.where(m, _splat(1), _splat(0)))
                ivs.append(ivec)
                cnts.append(plsc.all_reduce_population_count(m))
            base = off
            for j in range(GV):
                pos = base + plsc.cumsum(mis[j]) - mis[j]
                plsc.store_scatter(cand_ref, [pos], ivs[j],
                                   mask=mis[j] > 0)
                base = base + cnts[j]
            s2 = jnp.where(lane == ffs,
                           jnp.full((L,), _NEG_INF, jnp.float32), s)
            return (s2, base)

        return lax.fori_loop(0, nh, hit_body, (s, off))[1]

    off = lax.fori_loop(0, SB, pass_c, _splat(0))

    cnt = lax.reduce_max(off, axes=(0,))
    T = jnp.full((L,), _NEG_INF, jnp.float32)
    TI = jnp.zeros((L,), jnp.int32)

    def pass_d(k, c):
        T, TI = c
        iv = cand_ref[pl.ds(k * L, L)]
        valid = (_splat(k * L) + lane) < off
        ivc = jnp.minimum(jnp.maximum(iv, 0), N - 1)
        gv = plsc.load_gather(row_ref, [islot, ivc])
        gv = jnp.where(valid, gv, jnp.full((L,), _NEG_INF, jnp.float32))
        ivc = jnp.where(valid, ivc, _splat(0))
        nc = jnp.minimum(cnt - k * L, L)

        def ins_body(_, c2):
            gv, T, TI = c2
            ffs = plsc.all_reduce_ffs(gv >= t0)
            cv = _dyn_gather(gv, ffs)
            ci = _dyn_gather(ivc, ffs)
            icnt = plsc.all_reduce_population_count(T >= cv)
            shT = _dyn_gather(T, jnp.maximum(lane - 1, 0))
            shTI = _dyn_gather(TI, jnp.maximum(lane - 1, 0))
            T2 = jnp.where(lane < icnt, T, jnp.where(lane == icnt, cv, shT))
            TI2 = jnp.where(lane < icnt, TI,
                            jnp.where(lane == icnt, ci, shTI))
            gv2 = jnp.where(lane == ffs,
                            jnp.full((L,), _NEG_INF, jnp.float32), gv)
            return (gv2, T2, TI2)

        _, T, TI = lax.fori_loop(0, nc, ins_body, (gv, T, TI))
        return (T, TI)

    T, TI = lax.fori_loop(0, (cnt + L - 1) // L, pass_d, (T, TI))
    return T, TI


@functools.partial(
    pl.kernel,
    mesh=plsc.VectorSubcoreMesh(core_axis_name="c", subcore_axis_name="s"),
    compiler_params=pltpu.CompilerParams(needs_layout_passes=False,
                                         use_tc_tiling_on_sc=True),
    out_type=[
        jax.ShapeDtypeStruct((R, K), jnp.float32),
        jax.ShapeDtypeStruct((R, K), jnp.int32),
    ],
    scratch_types=[
        pltpu.VMEM((ROWS_PER_W, N), jnp.float32),
        pltpu.VMEM((N,), jnp.int32),
        pltpu.VMEM((GROUPS,), jnp.float32),
        pltpu.VMEM((K,), jnp.float32),
        pltpu.VMEM((K,), jnp.int32),
        pltpu.SemaphoreType.DMA,
        pltpu.SemaphoreType.DMA,
    ],
)
def _topk_kernel(scores_hbm, vals_hbm, idx_hbm,
                 rows_v, cand_v, summ_v, vals_v, idx_v, sem0, sem1):
    info = plsc.get_sparse_core_info()
    nc = info.num_cores
    wid = lax.axis_index("s") * nc + lax.axis_index("c")
    r0 = wid * ROWS_PER_W
    lane = lax.iota(jnp.int32, L)

    cp0 = pltpu.async_copy(scores_hbm.at[r0], rows_v.at[0], sem0)
    cp1 = pltpu.async_copy(scores_hbm.at[r0 + 1], rows_v.at[1], sem1)

    cp0.wait()
    T, TI = _process_row(rows_v, 0, cand_v, summ_v, lane)
    vals_v[...] = T
    idx_v[...] = TI
    pltpu.sync_copy(vals_v, vals_hbm.at[r0])
    pltpu.sync_copy(idx_v, idx_hbm.at[r0])

    cp1.wait()
    T, TI = _process_row(rows_v, 1, cand_v, summ_v, lane)
    vals_v[...] = T
    idx_v[...] = TI
    pltpu.sync_copy(vals_v, vals_hbm.at[r0 + 1])
    pltpu.sync_copy(idx_v, idx_hbm.at[r0 + 1])


def kernel(scores):
    vals, idx = _topk_kernel(scores)
    return vals, idx
